# Initial kernel scaffold; baseline (speedup 1.0000x reference)
import jax
import jax.numpy as jnp
from jax import lax
from jax.experimental import pallas as pl
from jax.experimental.pallas import tpu as pltpu
from jax.experimental.pallas import tpu_sc as plsc

F32 = jnp.float32
I32 = jnp.int32

HD = 128
ED = 16
G = 64

NC = 2
NS = 16
NW = NC * NS


def _relu(v):
    return jnp.maximum(v, 0.0)


def _dot(a, b):
    return jnp.dot(a, b, preferred_element_type=F32)


def _encode_body(x, encW, encb, wi, wj, h_out, ai_out, aj_out):
    h = _relu(_dot(x[...], encW[...]) + encb[...])
    h_out[...] = h
    ai_out[...] = _dot(h, wi[...])
    aj_out[...] = _dot(h, wj[...])


def _edge_body(g, ea, d2, we, wd, b1, w2, b2, m2_out):
    p = _dot(ea[...], we[...]) + d2[...] * wd[...] + b1[...]
    m1 = _relu(g[...] + p)
    m2_out[...] = _relu(_dot(m1, w2[...]) + b2[...])


def _node_body_a(h, a0, a1, w1h, w1a, b1, w2, b2, wi, wj,
                 hn_out, ai_out, aj_out):
    aggr = a0[...] + a1[...]
    u = _relu(_dot(h[...], w1h[...]) + _dot(aggr, w1a[...]) + b1[...])
    hn = _dot(u, w2[...]) + b2[...]
    hn_out[...] = hn
    ai_out[...] = _dot(hn, wi[...])
    aj_out[...] = _dot(hn, wj[...])


def _node_body(h, a0, a1, w1h, w1a, b1, w2, b2, hn_out):
    aggr = a0[...] + a1[...]
    u = _relu(_dot(h[...], w1h[...]) + _dot(aggr, w1a[...]) + b1[...])
    hn_out[...] = _dot(u, w2[...]) + b2[...]


def _pool_body(nblk, h, bi, rw1, rb1, rw2, rb2, rw3, rb3, out,
               ssum, smax, scnt):
    i = pl.program_id(0)

    @pl.when(i == 0)
    def _():
        ssum[...] = jnp.zeros_like(ssum)
        scnt[...] = jnp.zeros_like(scnt)
        smax[...] = jnp.full_like(smax, -3e38)

    hb = h[...]
    bb = bi[...]
    gids = lax.broadcasted_iota(I32, (1, G), 1)
    onehot = (bb == gids).astype(F32)
    dn = (((0,), (0,)), ((), ()))
    ssum[...] += lax.dot_general(onehot, hb, dn, preferred_element_type=F32)
    scnt[...] += lax.dot_general(onehot, jnp.ones_like(hb), dn,
                                 preferred_element_type=F32)
    gids3 = lax.broadcasted_iota(I32, (1, G, 1), 1)
    masked = jnp.where(bb[:, :, None] == gids3, hb[:, None, :], -3e38)
    smax[...] = jnp.maximum(smax[...], jnp.max(masked, axis=0))

    @pl.when(i == nblk - 1)
    def _():
        cnt = scnt[...]
        total = ssum[...]
        mean = total / jnp.maximum(cnt, 1.0)
        mx = jnp.where(cnt > 0, smax[...], 0.0)
        pooled = jnp.concatenate([mean, mx, total], axis=1)
        o = _relu(_dot(pooled, rw1[...]) + rb1[...])
        o = _relu(_dot(o, rw2[...]) + rb2[...])
        t = _dot(o, rw3[...]) + rb3[...]
        out[...] = jnp.maximum(t, 0.0) + jnp.log1p(jnp.exp(-jnp.abs(t)))


def _worker_id():
    return lax.axis_index("s") * NC + lax.axis_index("c")


def _dist2_body(epw, px, py, pz, srcr, dstr, out, pxv, pyv, pzv, sv, dv, ov):
    base = _worker_id() * epw
    pltpu.sync_copy(px, pxv)
    pltpu.sync_copy(py, pyv)
    pltpu.sync_copy(pz, pzv)
    pltpu.sync_copy(srcr.at[pl.ds(base, epw)], sv)
    pltpu.sync_copy(dstr.at[pl.ds(base, epw)], dv)

    def body(j, carry):
        si = sv[pl.ds(j * 16, 16)]
        di = dv[pl.ds(j * 16, 16)]
        dx = plsc.load_gather(pxv, [si]) - plsc.load_gather(pxv, [di])
        dy = plsc.load_gather(pyv, [si]) - plsc.load_gather(pyv, [di])
        dz = plsc.load_gather(pzv, [si]) - plsc.load_gather(pzv, [di])
        ov[pl.ds(j * 16, 16)] = dx * dx + dy * dy + dz * dz
        return carry

    lax.fori_loop(0, epw // 16, body, 0)
    pltpu.sync_copy(ov, out.at[pl.ds(base, epw)])


def _gather_body(epw, cg, ai, aj, srcr, dstr, g, isv, idv, rows, sem):
    base = _worker_id() * epw

    def chunk(ci, carry):
        off = base + ci * cg
        pltpu.sync_copy(dstr.at[pl.ds(off, cg)], idv)
        pltpu.sync_copy(srcr.at[pl.ds(off, cg)], isv)
        pltpu.async_copy(ai.at[idv], rows, sem).wait()
        pltpu.async_copy(aj.at[isv], rows, sem, add=True).wait()
        pltpu.sync_copy(rows, g.at[pl.ds(off, cg)])
        return carry

    lax.fori_loop(0, epw // cg, chunk, 0)


def _scatter_body(epw, cs, nps, m2, dstr, zrows, aggr_p, acc, idv, rows):
    c = lax.axis_index("c")
    s = lax.axis_index("s")
    base = (s * NC + c) * epw
    pltpu.sync_copy(zrows.at[pl.ds(s * nps, nps)], acc.at[pl.ds(s * nps, nps)])
    plsc.subcore_barrier()

    def chunk(ci, carry):
        off = base + ci * cs
        pltpu.sync_copy(dstr.at[pl.ds(off, cs)], idv)
        pltpu.sync_copy(m2.at[pl.ds(off, cs)], rows)
        pltpu.sync_copy(rows, acc.at[idv], add=True)
        return carry

    lax.fori_loop(0, epw // cs, chunk, 0)
    plsc.subcore_barrier()
    pltpu.sync_copy(acc.at[pl.ds(s * nps, nps)],
                    aggr_p.at[c, pl.ds(s * nps, nps)])


def kernel(x, pos, edge_index, edge_attr, batch, enc_W, enc_b, eW1, eb1,
           eW2, eb2, nW1, nb1, nW2, nb2, rW1, rb1, rW2, rb2, rW3, rb3):
    N = x.shape[0]
    E = edge_index.shape[1]
    L = eW1.shape[0]
    EPW = E // NW
    NPS = N // NS

    src = edge_index[0]
    dst = edge_index[1]

    mesh = plsc.VectorSubcoreMesh(core_axis_name="c", subcore_axis_name="s")

    d2 = pl.kernel(
        lambda *a: _dist2_body(EPW, *a),
        out_type=jax.ShapeDtypeStruct((E,), F32),
        mesh=mesh,
        scratch_types=[
            pltpu.VMEM((N,), F32), pltpu.VMEM((N,), F32), pltpu.VMEM((N,), F32),
            pltpu.VMEM((EPW,), I32), pltpu.VMEM((EPW,), I32),
            pltpu.VMEM((EPW,), F32),
        ],
    )(pos[:, 0], pos[:, 1], pos[:, 2], src, dst)

    BN = 1000
    gn = N // BN
    row_spec = pl.BlockSpec((BN, HD), lambda i: (i, 0))
    w_spec = pl.BlockSpec((HD, HD), lambda i: (0, 0))
    b_spec = pl.BlockSpec((1, HD), lambda i: (0, 0))
    h, ai, aj = pl.pallas_call(
        _encode_body,
        grid=(gn,),
        in_specs=[row_spec, w_spec, b_spec, w_spec, w_spec],
        out_specs=[row_spec, row_spec, row_spec],
        out_shape=[jax.ShapeDtypeStruct((N, HD), F32)] * 3,
    )(x, enc_W, enc_b.reshape(1, HD), eW1[0, :HD], eW1[0, HD:2 * HD])

    CG = 250
    CS = 250
    gather_call = pl.kernel(
        lambda *a: _gather_body(EPW, CG, *a),
        out_type=jax.ShapeDtypeStruct((E, HD), F32),
        mesh=mesh,
        scratch_types=[
            pltpu.VMEM((CG,), I32), pltpu.VMEM((CG,), I32),
            pltpu.VMEM((CG, HD), F32), pltpu.SemaphoreType.DMA,
        ],
    )
    scatter_call = pl.kernel(
        lambda *a: _scatter_body(EPW, CS, NPS, *a),
        out_type=jax.ShapeDtypeStruct((NC, N, HD), F32),
        mesh=mesh,
        scratch_types=[
            pltpu.VMEM_SHARED((N, HD), F32),
            pltpu.VMEM((CS,), I32), pltpu.VMEM((CS, HD), F32),
        ],
    )
    zrows = jnp.zeros((N, HD), F32)

    BE = 1280
    ge = E // BE
    erow = pl.BlockSpec((BE, HD), lambda i: (i, 0))
    edge_call = pl.pallas_call(
        _edge_body,
        grid=(ge,),
        in_specs=[
            erow,
            pl.BlockSpec((BE, ED), lambda i: (i, 0)),
            pl.BlockSpec((BE, 1), lambda i: (i, 0)),
            pl.BlockSpec((ED, HD), lambda i: (0, 0)),
            b_spec, b_spec, w_spec, b_spec,
        ],
        out_specs=erow,
        out_shape=jax.ShapeDtypeStruct((E, HD), F32),
    )

    node_call_a = pl.pallas_call(
        _node_body_a,
        grid=(gn,),
        in_specs=[row_spec, row_spec, row_spec,
                  w_spec, w_spec, b_spec, w_spec, b_spec, w_spec, w_spec],
        out_specs=[row_spec, row_spec, row_spec],
        out_shape=[jax.ShapeDtypeStruct((N, HD), F32)] * 3,
    )
    node_call = pl.pallas_call(
        _node_body,
        grid=(gn,),
        in_specs=[row_spec, row_spec, row_spec,
                  w_spec, w_spec, b_spec, w_spec, b_spec],
        out_specs=row_spec,
        out_shape=jax.ShapeDtypeStruct((N, HD), F32),
    )

    d2c = d2.reshape(E, 1)
    for l in range(L):
        g = gather_call(ai, aj, src, dst)
        m2 = edge_call(
            g, edge_attr, d2c,
            eW1[l, 2 * HD:2 * HD + ED],
            eW1[l, 2 * HD + ED:].reshape(1, HD),
            eb1[l].reshape(1, HD), eW2[l], eb2[l].reshape(1, HD))
        aggr_p = scatter_call(m2, dst, zrows)
        if l + 1 < L:
            h, ai, aj = node_call_a(
                h, aggr_p[0], aggr_p[1],
                nW1[l, :HD], nW1[l, HD:], nb1[l].reshape(1, HD),
                nW2[l], nb2[l].reshape(1, HD),
                eW1[l + 1, :HD], eW1[l + 1, HD:2 * HD])
        else:
            h = node_call(
                h, aggr_p[0], aggr_p[1],
                nW1[l, :HD], nW1[l, HD:], nb1[l].reshape(1, HD),
                nW2[l], nb2[l].reshape(1, HD))

    BP = 80
    gp = N // BP
    out = pl.pallas_call(
        lambda *a: _pool_body(gp, *a),
        grid=(gp,),
        in_specs=[
            pl.BlockSpec((BP, HD), lambda i: (i, 0)),
            pl.BlockSpec((BP, 1), lambda i: (i, 0)),
            pl.BlockSpec((3 * HD, HD), lambda i: (0, 0)),
            b_spec,
            pl.BlockSpec((HD, HD // 2), lambda i: (0, 0)),
            pl.BlockSpec((1, HD // 2), lambda i: (0, 0)),
            pl.BlockSpec((HD // 2, HD), lambda i: (0, 0)),
            b_spec,
        ],
        out_specs=pl.BlockSpec((G, HD), lambda i: (0, 0)),
        out_shape=jax.ShapeDtypeStruct((G, HD), F32),
        scratch_shapes=[pltpu.VMEM((G, HD), F32)] * 3,
    )(h, batch.reshape(N, 1), rW1, rb1.reshape(1, HD),
      rW2, rb2.reshape(1, HD // 2),
      jnp.pad(rW3, ((0, 0), (0, HD - 1))),
      jnp.broadcast_to(rb3, (1, HD)))
    return out[:, :1]


# speedup vs baseline: 2.8036x; 2.8036x over previous
import jax
import jax.numpy as jnp
from jax import lax
from jax.experimental import pallas as pl
from jax.experimental.pallas import tpu as pltpu
from jax.experimental.pallas import tpu_sc as plsc

F32 = jnp.float32
I32 = jnp.int32

HD = 128
ED = 16
G = 64
PD = 16

NC = 2
NS = 16
NW = NC * NS


def _relu(v):
    return jnp.maximum(v, 0.0)


def _dot(a, b):
    return jnp.dot(a, b, preferred_element_type=F32)


def _encode_body(x, encW, encb, wi, wj, pp, h_out, ai_out, aj_out):
    h = _relu(_dot(x[...], encW[...]) + encb[...])
    h_out[...] = h
    ai_out[...] = jnp.concatenate([_dot(h, wi[...]), -pp[...]], axis=1)
    aj_out[...] = jnp.concatenate([_dot(h, wj[...]), pp[...]], axis=1)


def _edge_body(g, ea, we, wd, b1, w2, b2, m2_out):
    gb = g[...]
    rel = gb[:, HD:]
    d2 = jnp.sum(rel * rel, axis=1, keepdims=True)
    p = _dot(ea[...], we[...]) + d2 * wd[...] + b1[...]
    m1 = _relu(gb[:, :HD] + p)
    m2_out[...] = _relu(_dot(m1, w2[...]) + b2[...])


def _node_body_a(h, a0, a1, w1h, w1a, b1, w2, b2, wi, wj, pp,
                 hn_out, ai_out, aj_out):
    aggr = a0[...] + a1[...]
    u = _relu(_dot(h[...], w1h[...]) + _dot(aggr, w1a[...]) + b1[...])
    hn = _dot(u, w2[...]) + b2[...]
    hn_out[...] = hn
    ai_out[...] = jnp.concatenate([_dot(hn, wi[...]), -pp[...]], axis=1)
    aj_out[...] = jnp.concatenate([_dot(hn, wj[...]), pp[...]], axis=1)


def _node_body(h, a0, a1, w1h, w1a, b1, w2, b2, hn_out):
    aggr = a0[...] + a1[...]
    u = _relu(_dot(h[...], w1h[...]) + _dot(aggr, w1a[...]) + b1[...])
    hn_out[...] = _dot(u, w2[...]) + b2[...]


def _pool_body(nblk, h, bi, rw1, rb1, rw2, rb2, rw3, rb3, out,
               ssum, smax, scnt):
    i = pl.program_id(0)

    @pl.when(i == 0)
    def _():
        ssum[...] = jnp.zeros_like(ssum)
        scnt[...] = jnp.zeros_like(scnt)
        smax[...] = jnp.full_like(smax, -3e38)

    hb = h[...]
    bb = bi[...]
    gids = lax.broadcasted_iota(I32, (1, G), 1)
    onehot = (bb == gids).astype(F32)
    dn = (((0,), (0,)), ((), ()))
    ssum[...] += lax.dot_general(onehot, hb, dn, preferred_element_type=F32)
    scnt[...] += lax.dot_general(onehot, jnp.ones_like(hb), dn,
                                 preferred_element_type=F32)
    gids3 = lax.broadcasted_iota(I32, (1, G, 1), 1)
    masked = jnp.where(bb[:, :, None] == gids3, hb[:, None, :], -3e38)
    smax[...] = jnp.maximum(smax[...], jnp.max(masked, axis=0))

    @pl.when(i == nblk - 1)
    def _():
        cnt = scnt[...]
        total = ssum[...]
        mean = total / jnp.maximum(cnt, 1.0)
        mx = jnp.where(cnt > 0, smax[...], 0.0)
        pooled = jnp.concatenate([mean, mx, total], axis=1)
        o = _relu(_dot(pooled, rw1[...]) + rb1[...])
        o = _relu(_dot(o, rw2[...]) + rb2[...])
        t = _dot(o, rw3[...]) + rb3[...]
        out[...] = jnp.maximum(t, 0.0) + jnp.log1p(jnp.exp(-jnp.abs(t)))


def _worker_id():
    return lax.axis_index("s") * NC + lax.axis_index("c")


def _gather_body(epw, cg, ai, aj, srcr, dstr, g, isv, idv, rows, sem):
    base = _worker_id() * epw

    def chunk(ci, carry):
        off = base + ci * cg
        pltpu.sync_copy(dstr.at[pl.ds(off, cg)], idv)
        pltpu.sync_copy(srcr.at[pl.ds(off, cg)], isv)
        pltpu.async_copy(ai.at[idv], rows, sem).wait()
        pltpu.async_copy(aj.at[isv], rows, sem, add=True).wait()
        pltpu.sync_copy(rows, g.at[pl.ds(off, cg)])
        return carry

    lax.fori_loop(0, epw // cg, chunk, 0)


def _scatter_body(epw, cs, nps, ntail, m2, dstr, zrows, aggr_p, acc,
                  idv, rows):
    c = lax.axis_index("c")
    s = lax.axis_index("s")
    base = (s * NC + c) * epw
    start = s * nps
    pltpu.sync_copy(zrows.at[pl.ds(start, nps)], acc.at[pl.ds(start, nps)])

    @pl.when(s == NS - 1)
    def _():
        pltpu.sync_copy(zrows.at[pl.ds(NS * nps, ntail)],
                        acc.at[pl.ds(NS * nps, ntail)])

    plsc.subcore_barrier()

    def chunk(ci, carry):
        off = base + ci * cs
        pltpu.sync_copy(dstr.at[pl.ds(off, cs)], idv)
        pltpu.sync_copy(m2.at[pl.ds(off, cs)], rows)
        pltpu.sync_copy(rows, acc.at[idv], add=True)
        return carry

    lax.fori_loop(0, epw // cs, chunk, 0)
    plsc.subcore_barrier()
    pltpu.sync_copy(acc.at[pl.ds(start, nps)],
                    aggr_p.at[c, pl.ds(start, nps)])

    @pl.when(s == NS - 1)
    def _():
        pltpu.sync_copy(acc.at[pl.ds(NS * nps, ntail)],
                        aggr_p.at[c, pl.ds(NS * nps, ntail)])


def kernel(x, pos, edge_index, edge_attr, batch, enc_W, enc_b, eW1, eb1,
           eW2, eb2, nW1, nb1, nW2, nb2, rW1, rb1, rW2, rb2, rW3, rb3):
    N = x.shape[0]
    E = edge_index.shape[1]
    L = eW1.shape[0]
    EPW = E // NW
    NPS = (N // NS) // 8 * 8
    NTAIL = N - NS * NPS

    src = edge_index[0]
    dst = edge_index[1]

    mesh = plsc.VectorSubcoreMesh(core_axis_name="c", subcore_axis_name="s",
                                  num_cores=NC, num_subcores=NS)

    pp = jnp.pad(pos, ((0, 0), (0, PD - pos.shape[1])))
    WT = HD + PD

    BN = 1000
    gn = N // BN
    row_spec = pl.BlockSpec((BN, HD), lambda i: (i, 0))
    tab_spec = pl.BlockSpec((BN, WT), lambda i: (i, 0))
    pp_spec = pl.BlockSpec((BN, PD), lambda i: (i, 0))
    w_spec = pl.BlockSpec((HD, HD), lambda i: (0, 0))
    b_spec = pl.BlockSpec((1, HD), lambda i: (0, 0))
    tab_shape = jax.ShapeDtypeStruct((N, WT), F32)
    h, ai, aj = pl.pallas_call(
        _encode_body,
        grid=(gn,),
        in_specs=[row_spec, w_spec, b_spec, w_spec, w_spec, pp_spec],
        out_specs=[row_spec, tab_spec, tab_spec],
        out_shape=[jax.ShapeDtypeStruct((N, HD), F32), tab_shape, tab_shape],
    )(x, enc_W, enc_b.reshape(1, HD), eW1[0, :HD], eW1[0, HD:2 * HD], pp)

    CG = 400
    CS = 200
    gather_call = pl.kernel(
        lambda *a: _gather_body(EPW, CG, *a),
        out_type=jax.ShapeDtypeStruct((E, WT), F32),
        mesh=mesh,
        compiler_params=pltpu.CompilerParams(use_tc_tiling_on_sc=False),
        scratch_types=[
            pltpu.VMEM((CG,), I32), pltpu.VMEM((CG,), I32),
            pltpu.VMEM((CG, WT), F32), pltpu.SemaphoreType.DMA,
        ],
    )
    scatter_call = pl.kernel(
        lambda *a: _scatter_body(EPW, CS, NPS, NTAIL, *a),
        out_type=jax.ShapeDtypeStruct((NC, N, HD), F32),
        mesh=mesh,
        scratch_types=[
            pltpu.VMEM_SHARED((N, HD), F32),
            pltpu.VMEM((CS,), I32), pltpu.VMEM((CS, HD), F32),
        ],
    )
    zrows = jnp.zeros((N, HD), F32)

    BE = 1280
    ge = E // BE
    edge_call = pl.pallas_call(
        _edge_body,
        grid=(ge,),
        in_specs=[
            pl.BlockSpec((BE, WT), lambda i: (i, 0)),
            pl.BlockSpec((BE, ED), lambda i: (i, 0)),
            pl.BlockSpec((ED, HD), lambda i: (0, 0)),
            b_spec, b_spec, w_spec, b_spec,
        ],
        out_specs=pl.BlockSpec((BE, HD), lambda i: (i, 0)),
        out_shape=jax.ShapeDtypeStruct((E, HD), F32),
    )

    node_call_a = pl.pallas_call(
        _node_body_a,
        grid=(gn,),
        in_specs=[row_spec, row_spec, row_spec,
                  w_spec, w_spec, b_spec, w_spec, b_spec, w_spec, w_spec,
                  pp_spec],
        out_specs=[row_spec, tab_spec, tab_spec],
        out_shape=[jax.ShapeDtypeStruct((N, HD), F32), tab_shape, tab_shape],
    )
    node_call = pl.pallas_call(
        _node_body,
        grid=(gn,),
        in_specs=[row_spec, row_spec, row_spec,
                  w_spec, w_spec, b_spec, w_spec, b_spec],
        out_specs=row_spec,
        out_shape=jax.ShapeDtypeStruct((N, HD), F32),
    )

    for l in range(L):
        g = gather_call(ai, aj, src, dst)
        m2 = edge_call(
            g, edge_attr,
            eW1[l, 2 * HD:2 * HD + ED],
            eW1[l, 2 * HD + ED:].reshape(1, HD),
            eb1[l].reshape(1, HD), eW2[l], eb2[l].reshape(1, HD))
        aggr_p = scatter_call(m2, dst, zrows)
        if l + 1 < L:
            h, ai, aj = node_call_a(
                h, aggr_p[0], aggr_p[1],
                nW1[l, :HD], nW1[l, HD:], nb1[l].reshape(1, HD),
                nW2[l], nb2[l].reshape(1, HD),
                eW1[l + 1, :HD], eW1[l + 1, HD:2 * HD], pp)
        else:
            h = node_call(
                h, aggr_p[0], aggr_p[1],
                nW1[l, :HD], nW1[l, HD:], nb1[l].reshape(1, HD),
                nW2[l], nb2[l].reshape(1, HD))

    BP = 80
    gp = N // BP
    out = pl.pallas_call(
        lambda *a: _pool_body(gp, *a),
        grid=(gp,),
        in_specs=[
            pl.BlockSpec((BP, HD), lambda i: (i, 0)),
            pl.BlockSpec((BP, 1), lambda i: (i, 0)),
            pl.BlockSpec((3 * HD, HD), lambda i: (0, 0)),
            b_spec,
            pl.BlockSpec((HD, HD // 2), lambda i: (0, 0)),
            pl.BlockSpec((1, HD // 2), lambda i: (0, 0)),
            pl.BlockSpec((HD // 2, HD), lambda i: (0, 0)),
            b_spec,
        ],
        out_specs=pl.BlockSpec((G, HD), lambda i: (0, 0)),
        out_shape=jax.ShapeDtypeStruct((G, HD), F32),
        scratch_shapes=[pltpu.VMEM((G, HD), F32)] * 3,
    )(h, batch.reshape(N, 1), rW1, rb1.reshape(1, HD),
      rW2, rb2.reshape(1, HD // 2),
      jnp.pad(rW3, ((0, 0), (0, HD - 1))),
      jnp.broadcast_to(rb3, (1, HD)))
    return out[:, :1]


# speedup vs baseline: 3.5189x; 1.2551x over previous
import jax
import jax.numpy as jnp
from jax import lax
from jax.experimental import pallas as pl
from jax.experimental.pallas import tpu as pltpu
from jax.experimental.pallas import tpu_sc as plsc

F32 = jnp.float32
I32 = jnp.int32

HD = 128
ED = 16
G = 64

NC = 2
NS = 16
NW = NC * NS


def _relu(v):
    return jnp.maximum(v, 0.0)


def _dot(a, b):
    return jnp.dot(a, b, preferred_element_type=F32)


def _encode_body(x, encW, encb, h_out):
    h_out[...] = _relu(_dot(x[...], encW[...]) + encb[...])


def _edge_body(ga, gb, ea, d2, w01, wt, b1, w2, b2, m2_out):
    xij = jnp.concatenate([ga[...], gb[...]], axis=1)
    tail = jnp.concatenate([ea[...], d2[...]], axis=1)
    m1 = _relu(_dot(xij, w01[...]) + _dot(tail, wt[...]) + b1[...])
    m2_out[...] = _relu(_dot(m1, w2[...]) + b2[...])


def _node_body(h, a0, a1, w1, b1, w2, b2, hn_out):
    u = jnp.concatenate([h[...], a0[...] + a1[...]], axis=1)
    u = _relu(_dot(u, w1[...]) + b1[...])
    hn_out[...] = _dot(u, w2[...]) + b2[...]


def _pool_body(nblk, h, bi, rw1, rb1, rw2, rb2, rw3, rb3, out,
               ssum, smax, scnt):
    i = pl.program_id(0)

    @pl.when(i == 0)
    def _():
        ssum[...] = jnp.zeros_like(ssum)
        scnt[...] = jnp.zeros_like(scnt)
        smax[...] = jnp.full_like(smax, -3e38)

    hb = h[...]
    bb = bi[...]
    gids = lax.broadcasted_iota(I32, (1, G), 1)
    onehot = (bb == gids).astype(F32)
    dn = (((0,), (0,)), ((), ()))
    ssum[...] += lax.dot_general(onehot, hb, dn, preferred_element_type=F32,
                                 precision=lax.Precision.HIGHEST)
    scnt[...] += lax.dot_general(onehot, jnp.ones_like(hb), dn,
                                 preferred_element_type=F32,
                                 precision=lax.Precision.HIGHEST)
    gids3 = lax.broadcasted_iota(I32, (1, G, 1), 1)
    masked = jnp.where(bb[:, :, None] == gids3, hb[:, None, :], -3e38)
    smax[...] = jnp.maximum(smax[...], jnp.max(masked, axis=0))

    @pl.when(i == nblk - 1)
    def _():
        cnt = scnt[...]
        total = ssum[...]
        mean = total / jnp.maximum(cnt, 1.0)
        mx = jnp.where(cnt > 0, smax[...], 0.0)
        pooled = jnp.concatenate([mean, mx, total], axis=1)
        o = _relu(_dot(pooled, rw1[...]) + rb1[...])
        o = _relu(_dot(o, rw2[...]) + rb2[...])
        t = _dot(o, rw3[...]) + rb3[...]
        out[...] = jnp.maximum(t, 0.0) + jnp.log1p(jnp.exp(-jnp.abs(t)))


def _worker_id():
    return lax.axis_index("s") * NC + lax.axis_index("c")


def _dist2_body(epw, px, py, pz, srcr, dstr, out, pxv, pyv, pzv, sv, dv, ov):
    base = _worker_id() * epw
    pltpu.sync_copy(px, pxv)
    pltpu.sync_copy(py, pyv)
    pltpu.sync_copy(pz, pzv)
    pltpu.sync_copy(srcr.at[pl.ds(base, epw)], sv)
    pltpu.sync_copy(dstr.at[pl.ds(base, epw)], dv)

    def body(j, carry):
        si = sv[pl.ds(j * 16, 16)]
        di = dv[pl.ds(j * 16, 16)]
        dx = plsc.load_gather(pxv, [si]) - plsc.load_gather(pxv, [di])
        dy = plsc.load_gather(pyv, [si]) - plsc.load_gather(pyv, [di])
        dz = plsc.load_gather(pzv, [si]) - plsc.load_gather(pzv, [di])
        ov[pl.ds(j * 16, 16)] = dx * dx + dy * dy + dz * dz
        return carry

    lax.fori_loop(0, epw // 16, body, 0)
    pltpu.sync_copy(ov, out.at[pl.ds(base, epw)])


def _gather_body(epw, cg, tab, srcr, dstr, ga, gb, isv, idv,
                 rowsa, rowsb, sema, semb):
    base = _worker_id() * epw

    def chunk(ci, carry):
        off = base + ci * cg
        pltpu.sync_copy(dstr.at[pl.ds(off, cg)], idv)
        pltpu.sync_copy(srcr.at[pl.ds(off, cg)], isv)
        cpa = pltpu.async_copy(tab.at[idv], rowsa, sema)
        cpb = pltpu.async_copy(tab.at[isv], rowsb, semb)
        cpa.wait()
        pltpu.sync_copy(rowsa, ga.at[pl.ds(off, cg)])
        cpb.wait()
        pltpu.sync_copy(rowsb, gb.at[pl.ds(off, cg)])
        return carry

    lax.fori_loop(0, epw // cg, chunk, 0)


def _scatter_body(epw, cs, nps, ntail, m2, dstr, zrows, aggr_p, acc,
                  idv, rows):
    c = lax.axis_index("c")
    s = lax.axis_index("s")
    base = (s * NC + c) * epw
    start = s * nps
    pltpu.sync_copy(zrows.at[pl.ds(start, nps)], acc.at[pl.ds(start, nps)])

    @pl.when(s == NS - 1)
    def _():
        pltpu.sync_copy(zrows.at[pl.ds(NS * nps, ntail)],
                        acc.at[pl.ds(NS * nps, ntail)])

    plsc.subcore_barrier()

    def chunk(ci, carry):
        off = base + ci * cs
        pltpu.sync_copy(dstr.at[pl.ds(off, cs)], idv)
        pltpu.sync_copy(m2.at[pl.ds(off, cs)], rows)
        pltpu.sync_copy(rows, acc.at[idv], add=True)
        return carry

    lax.fori_loop(0, epw // cs, chunk, 0)
    plsc.subcore_barrier()
    pltpu.sync_copy(acc.at[pl.ds(start, nps)],
                    aggr_p.at[c, pl.ds(start, nps)])

    @pl.when(s == NS - 1)
    def _():
        pltpu.sync_copy(acc.at[pl.ds(NS * nps, ntail)],
                        aggr_p.at[c, pl.ds(NS * nps, ntail)])


def kernel(x, pos, edge_index, edge_attr, batch, enc_W, enc_b, eW1, eb1,
           eW2, eb2, nW1, nb1, nW2, nb2, rW1, rb1, rW2, rb2, rW3, rb3):
    N = x.shape[0]
    E = edge_index.shape[1]
    L = eW1.shape[0]
    EPW = E // NW
    NPS = (N // NS) // 8 * 8
    NTAIL = N - NS * NPS

    src = edge_index[0]
    dst = edge_index[1]

    mesh = plsc.VectorSubcoreMesh(core_axis_name="c", subcore_axis_name="s",
                                  num_cores=NC, num_subcores=NS)

    d2 = pl.kernel(
        lambda *a: _dist2_body(EPW, *a),
        out_type=jax.ShapeDtypeStruct((E,), F32),
        mesh=mesh,
        compiler_params=pltpu.CompilerParams(needs_layout_passes=False),
        scratch_types=[
            pltpu.VMEM((N,), F32), pltpu.VMEM((N,), F32),
            pltpu.VMEM((N,), F32),
            pltpu.VMEM((EPW,), I32), pltpu.VMEM((EPW,), I32),
            pltpu.VMEM((EPW,), F32),
        ],
    )(pos[:, 0], pos[:, 1], pos[:, 2], src, dst)

    BN = 1000
    gn = N // BN
    row_spec = pl.BlockSpec((BN, HD), lambda i: (i, 0))
    w_spec = pl.BlockSpec((HD, HD), lambda i: (0, 0))
    b_spec = pl.BlockSpec((1, HD), lambda i: (0, 0))
    h = pl.pallas_call(
        _encode_body,
        grid=(gn,),
        in_specs=[row_spec, w_spec, b_spec],
        out_specs=row_spec,
        out_shape=jax.ShapeDtypeStruct((N, HD), F32),
    )(x, enc_W, enc_b.reshape(1, HD))

    CG = 400
    CS = 200
    gather_call = pl.kernel(
        lambda *a: _gather_body(EPW, CG, *a),
        out_type=[jax.ShapeDtypeStruct((E, HD), F32)] * 2,
        mesh=mesh,
        scratch_types=[
            pltpu.VMEM((CG,), I32), pltpu.VMEM((CG,), I32),
            pltpu.VMEM((CG, HD), F32), pltpu.VMEM((CG, HD), F32),
            pltpu.SemaphoreType.DMA, pltpu.SemaphoreType.DMA,
        ],
    )
    scatter_call = pl.kernel(
        lambda *a: _scatter_body(EPW, CS, NPS, NTAIL, *a),
        out_type=jax.ShapeDtypeStruct((NC, N, HD), F32),
        mesh=mesh,
        scratch_types=[
            pltpu.VMEM_SHARED((N, HD), F32),
            pltpu.VMEM((CS,), I32), pltpu.VMEM((CS, HD), F32),
        ],
    )
    zrows = jnp.zeros((N, HD), F32)

    BE = 1280
    ge = E // BE
    erow = pl.BlockSpec((BE, HD), lambda i: (i, 0))
    edge_call = pl.pallas_call(
        _edge_body,
        grid=(ge,),
        in_specs=[
            erow, erow,
            pl.BlockSpec((BE, ED), lambda i: (i, 0)),
            pl.BlockSpec((BE, 1), lambda i: (i, 0)),
            pl.BlockSpec((2 * HD, HD), lambda i: (0, 0)),
            pl.BlockSpec((ED + 1, HD), lambda i: (0, 0)),
            b_spec, w_spec, b_spec,
        ],
        out_specs=erow,
        out_shape=jax.ShapeDtypeStruct((E, HD), F32),
    )
    d2c = d2.reshape(E, 1)

    node_call = pl.pallas_call(
        _node_body,
        grid=(gn,),
        in_specs=[row_spec, row_spec, row_spec,
                  pl.BlockSpec((2 * HD, HD), lambda i: (0, 0)),
                  b_spec, w_spec, b_spec],
        out_specs=row_spec,
        out_shape=jax.ShapeDtypeStruct((N, HD), F32),
    )

    for l in range(L):
        ga, gb = gather_call(h, src, dst)
        m2 = edge_call(
            ga, gb, edge_attr, d2c,
            eW1[l, :2 * HD], eW1[l, 2 * HD:],
            eb1[l].reshape(1, HD), eW2[l], eb2[l].reshape(1, HD))
        aggr_p = scatter_call(m2, dst, zrows)
        h = node_call(
            h, aggr_p[0], aggr_p[1],
            nW1[l], nb1[l].reshape(1, HD),
            nW2[l], nb2[l].reshape(1, HD))

    BP = 80
    gp = N // BP
    out = pl.pallas_call(
        lambda *a: _pool_body(gp, *a),
        grid=(gp,),
        in_specs=[
            pl.BlockSpec((BP, HD), lambda i: (i, 0)),
            pl.BlockSpec((BP, 1), lambda i: (i, 0)),
            pl.BlockSpec((3 * HD, HD), lambda i: (0, 0)),
            b_spec,
            pl.BlockSpec((HD, HD // 2), lambda i: (0, 0)),
            pl.BlockSpec((1, HD // 2), lambda i: (0, 0)),
            pl.BlockSpec((HD // 2, HD), lambda i: (0, 0)),
            b_spec,
        ],
        out_specs=pl.BlockSpec((G, HD), lambda i: (0, 0)),
        out_shape=jax.ShapeDtypeStruct((G, HD), F32),
        scratch_shapes=[pltpu.VMEM((G, HD), F32)] * 3,
    )(h, batch.reshape(N, 1), rW1, rb1.reshape(1, HD),
      rW2, rb2.reshape(1, HD // 2),
      jnp.pad(rW3, ((0, 0), (0, HD - 1))),
      jnp.broadcast_to(rb3, (1, HD)))
    return out[:, :1]
